# baseline (speedup 1.0000x reference)
import jax
import jax.numpy as jnp
from jax.experimental import pallas as pl
from jax.experimental.pallas import tpu as pltpu

H = 16
D = 128
CB = 32
SB = 64
WIN = 512
TOPN = 4
S = 2048
HID = 2048
NC = S // CB
NS = S // SB
NQ = NS
NW = S // WIN
SCALE = 1.0 / (D ** 0.5)
_NEG = -1e9

_T_DN = (((1,), (1,)), ((), ()))


def _mm_body(a_ref, b_ref, o_ref):
    a = a_ref[...].astype(jnp.bfloat16)
    b = b_ref[...].astype(jnp.bfloat16)
    o_ref[...] = jnp.dot(a, b, preferred_element_type=jnp.float32)


def _matmul(a, b, bm, bn):
    m, k = a.shape
    _, n = b.shape
    return pl.pallas_call(
        _mm_body,
        grid=(m // bm, n // bn),
        in_specs=[
            pl.BlockSpec((bm, k), lambda i, j: (i, 0)),
            pl.BlockSpec((k, bn), lambda i, j: (0, j)),
        ],
        out_specs=pl.BlockSpec((bm, bn), lambda i, j: (i, j)),
        out_shape=jax.ShapeDtypeStruct((m, n), jnp.float32),
    )(a, b)


def _cmp_body(q_ref, k_ref, v_ref, oc_ref, u_ref):
    h = pl.program_id(0)
    q = q_ref[...]
    k = k_ref[...]
    v = v_ref[...]
    kc = jnp.mean(k.reshape(NC, CB, D), axis=1)
    vc = jnp.mean(v.reshape(NC, CB, D), axis=1)
    s = jax.lax.dot_general(
        q.astype(jnp.bfloat16), kc.astype(jnp.bfloat16), _T_DN,
        preferred_element_type=jnp.float32) * SCALE
    t = jax.lax.broadcasted_iota(jnp.int32, (S, NC), 0)
    n = jax.lax.broadcasted_iota(jnp.int32, (S, NC), 1)
    s = jnp.where(n * CB <= t, s, _NEG)
    m = jnp.max(s, axis=1, keepdims=True)
    e = jnp.exp(s - m)
    p = e / jnp.sum(e, axis=1, keepdims=True)
    oc_ref[...] = jnp.dot(p.astype(jnp.bfloat16), vc.astype(jnp.bfloat16),
                          preferred_element_type=jnp.float32)
    u = jnp.sum(p.reshape(NQ, SB, NC), axis=1)

    @pl.when(h == 0)
    def _():
        u_ref[...] = u

    @pl.when(h > 0)
    def _():
        u_ref[...] += u


def _compressed(qkv):
    return pl.pallas_call(
        _cmp_body,
        grid=(H,),
        in_specs=[
            pl.BlockSpec((S, D), lambda h: (0, h)),
            pl.BlockSpec((S, D), lambda h: (0, H + h)),
            pl.BlockSpec((S, D), lambda h: (0, 2 * H + h)),
        ],
        out_specs=[
            pl.BlockSpec((S, D), lambda h: (0, h)),
            pl.BlockSpec((NQ, NC), lambda h: (0, 0)),
        ],
        out_shape=[
            jax.ShapeDtypeStruct((S, HID), jnp.float32),
            jax.ShapeDtypeStruct((NQ, NC), jnp.float32),
        ],
    )(qkv, qkv, qkv)


def _topk_body(u_ref, sel_ref):
    u = u_ref[...]
    ut = u.T
    ub = jnp.sum(ut.reshape(NS, 2, NQ), axis=1)
    imp = ub.T
    r = jax.lax.broadcasted_iota(jnp.int32, (NQ, NS), 0)
    c = jax.lax.broadcasted_iota(jnp.int32, (NQ, NS), 1)
    ssc = jnp.where(c <= r, imp, -1e30)
    ssc = (ssc + 1e20 * (c == r).astype(jnp.float32)
           + 1e19 * (c == 0).astype(jnp.float32))
    out = jnp.zeros((NQ, 128), jnp.int32)
    colp = jax.lax.broadcasted_iota(jnp.int32, (NQ, 128), 1)
    for ti in range(TOPN):
        mx = jnp.max(ssc, axis=1, keepdims=True)
        idx = jnp.min(jnp.where(ssc >= mx, c, NS), axis=1, keepdims=True)
        out = out + jnp.where(colp == ti, idx, 0)
        ssc = jnp.where(c == idx, -jnp.inf, ssc)
    sel_ref[...] = out


def _topk(u):
    return pl.pallas_call(
        _topk_body,
        out_shape=jax.ShapeDtypeStruct((NQ, 128), jnp.int32),
    )(u)


def _fine_body(sel_ref, q_ref, k0, k1, k2, k3, v0, v1, v2, v3, o_ref):
    i = pl.program_id(0)
    q = q_ref[...].astype(jnp.bfloat16)
    r = jax.lax.broadcasted_iota(jnp.int32, (SB, SB), 0)
    c = jax.lax.broadcasted_iota(jnp.int32, (SB, SB), 1)
    q_pos = i * SB + r
    parts = []
    for n, kr in enumerate((k0, k1, k2, k3)):
        sel = sel_ref[i * TOPN + n]
        s = jax.lax.dot_general(q, kr[...].astype(jnp.bfloat16), _T_DN,
                                preferred_element_type=jnp.float32) * SCALE
        kv_pos = sel * SB + c
        parts.append(jnp.where(kv_pos <= q_pos, s, _NEG))
    s = jnp.concatenate(parts, axis=1)
    m = jnp.max(s, axis=1, keepdims=True)
    e = jnp.exp(s - m)
    p = (e / jnp.sum(e, axis=1, keepdims=True)).astype(jnp.bfloat16)
    acc = jnp.zeros((SB, D), jnp.float32)
    for n, vr in enumerate((v0, v1, v2, v3)):
        acc += jnp.dot(p[:, n * SB:(n + 1) * SB], vr[...].astype(jnp.bfloat16),
                       preferred_element_type=jnp.float32)
    o_ref[...] = acc


def _fine(qkv, sel):
    def kmap(n):
        return lambda i, h, sref: (sref[i * TOPN + n], H + h)

    def vmap_(n):
        return lambda i, h, sref: (sref[i * TOPN + n], 2 * H + h)

    grid_spec = pltpu.PrefetchScalarGridSpec(
        num_scalar_prefetch=1,
        grid=(NQ, H),
        in_specs=[
            pl.BlockSpec((SB, D), lambda i, h, sref: (i, h)),
            pl.BlockSpec((SB, D), kmap(0)),
            pl.BlockSpec((SB, D), kmap(1)),
            pl.BlockSpec((SB, D), kmap(2)),
            pl.BlockSpec((SB, D), kmap(3)),
            pl.BlockSpec((SB, D), vmap_(0)),
            pl.BlockSpec((SB, D), vmap_(1)),
            pl.BlockSpec((SB, D), vmap_(2)),
            pl.BlockSpec((SB, D), vmap_(3)),
        ],
        out_specs=pl.BlockSpec((SB, D), lambda i, h, sref: (i, h)),
    )
    return pl.pallas_call(
        _fine_body,
        grid_spec=grid_spec,
        out_shape=jax.ShapeDtypeStruct((S, HID), jnp.float32),
    )(sel, qkv, qkv, qkv, qkv, qkv, qkv, qkv, qkv, qkv)


def _wc_body(q_ref, kp_ref, kc_ref, vp_ref, vc_ref, oc_ref, os_ref, g_ref,
             o_ref):
    w = pl.program_id(0)
    q = q_ref[...].astype(jnp.bfloat16)
    sp = jax.lax.dot_general(q, kp_ref[...].astype(jnp.bfloat16), _T_DN,
                             preferred_element_type=jnp.float32) * SCALE
    sc_ = jax.lax.dot_general(q, kc_ref[...].astype(jnp.bfloat16), _T_DN,
                              preferred_element_type=jnp.float32) * SCALE
    r = jax.lax.broadcasted_iota(jnp.int32, (WIN, WIN), 0)
    c = jax.lax.broadcasted_iota(jnp.int32, (WIN, WIN), 1)
    sp = jnp.where((r < c) & (w > 0), sp, _NEG)
    sc_ = jnp.where(c <= r, sc_, _NEG)
    m = jnp.maximum(jnp.max(sp, axis=1, keepdims=True),
                    jnp.max(sc_, axis=1, keepdims=True))
    ep = jnp.exp(sp - m)
    ec = jnp.exp(sc_ - m)
    den = jnp.sum(ep, axis=1, keepdims=True) + jnp.sum(ec, axis=1, keepdims=True)
    pp = (ep / den).astype(jnp.bfloat16)
    pc = (ec / den).astype(jnp.bfloat16)
    ow = (jnp.dot(pp, vp_ref[...].astype(jnp.bfloat16),
                  preferred_element_type=jnp.float32)
          + jnp.dot(pc, vc_ref[...].astype(jnp.bfloat16),
                    preferred_element_type=jnp.float32))
    h = pl.program_id(1)
    g = g_ref[...]
    g = 1.0 / (1.0 + jnp.exp(-g))
    lane = jax.lax.broadcasted_iota(jnp.int32, (WIN, 3 * H), 1)

    def gcol(j):
        return jnp.sum(jnp.where(lane == 3 * h + j, g, 0.0), axis=1,
                       keepdims=True)

    o_ref[...] = (gcol(0) * oc_ref[...] + gcol(1) * os_ref[...]
                  + gcol(2) * ow)


def _win_combine(qkv, o_cmp, o_sel, graw):
    def prev_map(w, h, off):
        return (jnp.where(w == 0, 0, w - 1), off + h)

    return pl.pallas_call(
        _wc_body,
        grid=(NW, H),
        in_specs=[
            pl.BlockSpec((WIN, D), lambda w, h: (w, h)),
            pl.BlockSpec((WIN, D), lambda w, h: prev_map(w, h, H)),
            pl.BlockSpec((WIN, D), lambda w, h: (w, H + h)),
            pl.BlockSpec((WIN, D), lambda w, h: prev_map(w, h, 2 * H)),
            pl.BlockSpec((WIN, D), lambda w, h: (w, 2 * H + h)),
            pl.BlockSpec((WIN, D), lambda w, h: (w, h)),
            pl.BlockSpec((WIN, D), lambda w, h: (w, h)),
            pl.BlockSpec((WIN, 3 * H), lambda w, h: (w, 0)),
        ],
        out_specs=pl.BlockSpec((WIN, D), lambda w, h: (w, h)),
        out_shape=jax.ShapeDtypeStruct((S, HID), jnp.float32),
    )(qkv, qkv, qkv, qkv, qkv, o_cmp, o_sel, graw)


def kernel(x, Wq, Wk, Wv, Wg, Wo):
    x2 = x.reshape(S, HID)
    w3 = jnp.concatenate([Wq, Wk, Wv], axis=1)
    qkv = _matmul(x2, w3, 512, 512)
    graw = _matmul(x2, Wg, 512, 48)
    o_cmp, u = _compressed(qkv)
    selp = _topk(u)
    sel = selp[:, :TOPN].reshape(-1).astype(jnp.int32)
    o_sel = _fine(qkv, sel)
    o_cmb = _win_combine(qkv, o_cmp, o_sel, graw)
    out = _matmul(o_cmb, Wo, 256, 512)
    return out.reshape(1, S, HID)


# speedup vs baseline: 1.4715x; 1.4715x over previous
import jax
import jax.numpy as jnp
from jax.experimental import pallas as pl
from jax.experimental.pallas import tpu as pltpu

H = 16
D = 128
CB = 32
SB = 64
WIN = 512
TOPN = 4
S = 2048
HID = 2048
NC = S // CB
NS = S // SB
NQ = NS
NW = S // WIN
SCALE = 1.0 / (D ** 0.5)
_NEG = -1e9

_T_DN = (((1,), (1,)), ((), ()))


def _mm_body(a_ref, b_ref, o_ref):
    a = a_ref[...].astype(jnp.bfloat16)
    b = b_ref[...].astype(jnp.bfloat16)
    o_ref[...] = jnp.dot(a, b, preferred_element_type=jnp.float32)


def _matmul(a, b, bm, bn):
    m, k = a.shape
    _, n = b.shape
    return pl.pallas_call(
        _mm_body,
        grid=(m // bm, n // bn),
        in_specs=[
            pl.BlockSpec((bm, k), lambda i, j: (i, 0)),
            pl.BlockSpec((k, bn), lambda i, j: (0, j)),
        ],
        out_specs=pl.BlockSpec((bm, bn), lambda i, j: (i, j)),
        out_shape=jax.ShapeDtypeStruct((m, n), jnp.float32),
    )(a, b)


def _cmp_body(q_ref, k_ref, v_ref, oc_ref, u_ref):
    h = pl.program_id(0)
    q = q_ref[...]
    k = k_ref[...]
    v = v_ref[...]
    kc = jnp.mean(k.reshape(NC, CB, D), axis=1)
    vc = jnp.mean(v.reshape(NC, CB, D), axis=1)
    s = jax.lax.dot_general(
        q.astype(jnp.bfloat16), kc.astype(jnp.bfloat16), _T_DN,
        preferred_element_type=jnp.float32) * SCALE
    t = jax.lax.broadcasted_iota(jnp.int32, (S, NC), 0)
    n = jax.lax.broadcasted_iota(jnp.int32, (S, NC), 1)
    s = jnp.where(n * CB <= t, s, _NEG)
    m = jnp.max(s, axis=1, keepdims=True)
    e = jnp.exp(s - m)
    p = e / jnp.sum(e, axis=1, keepdims=True)
    oc_ref[...] = jnp.dot(p.astype(jnp.bfloat16), vc.astype(jnp.bfloat16),
                          preferred_element_type=jnp.float32)
    u = jnp.sum(p.reshape(NQ, SB, NC), axis=1)

    @pl.when(h == 0)
    def _():
        u_ref[...] = u

    @pl.when(h > 0)
    def _():
        u_ref[...] += u


def _compressed(qkv):
    return pl.pallas_call(
        _cmp_body,
        grid=(H,),
        in_specs=[
            pl.BlockSpec((S, D), lambda h: (0, h)),
            pl.BlockSpec((S, D), lambda h: (0, H + h)),
            pl.BlockSpec((S, D), lambda h: (0, 2 * H + h)),
        ],
        out_specs=[
            pl.BlockSpec((S, D), lambda h: (0, h)),
            pl.BlockSpec((NQ, NC), lambda h: (0, 0)),
        ],
        out_shape=[
            jax.ShapeDtypeStruct((S, HID), jnp.float32),
            jax.ShapeDtypeStruct((NQ, NC), jnp.float32),
        ],
    )(qkv, qkv, qkv)


def _topk_body(u_ref, sel_ref):
    u = u_ref[...]
    ut = u.T
    ub = jnp.sum(ut.reshape(NS, 2, NQ), axis=1)
    imp = ub.T
    r = jax.lax.broadcasted_iota(jnp.int32, (NQ, NS), 0)
    c = jax.lax.broadcasted_iota(jnp.int32, (NQ, NS), 1)
    ssc = jnp.where(c <= r, imp, -1e30)
    ssc = (ssc + 1e20 * (c == r).astype(jnp.float32)
           + 1e19 * (c == 0).astype(jnp.float32))
    out = jnp.zeros((NQ, 128), jnp.int32)
    colp = jax.lax.broadcasted_iota(jnp.int32, (NQ, 128), 1)
    for ti in range(TOPN):
        mx = jnp.max(ssc, axis=1, keepdims=True)
        idx = jnp.min(jnp.where(ssc >= mx, c, NS), axis=1, keepdims=True)
        out = out + jnp.where(colp == ti, idx, 0)
        ssc = jnp.where(c == idx, -jnp.inf, ssc)
    sel_ref[...] = out


def _topk(u):
    return pl.pallas_call(
        _topk_body,
        out_shape=jax.ShapeDtypeStruct((NQ, 128), jnp.int32),
    )(u)


def _fine_body(sel_ref, q_ref, k0, k1, k2, k3, v0, v1, v2, v3, o_ref):
    i = pl.program_id(0)
    r = jax.lax.broadcasted_iota(jnp.int32, (SB, TOPN * SB), 0)
    c = jax.lax.broadcasted_iota(jnp.int32, (SB, TOPN * SB), 1)
    n_of_c = c // SB
    sel0 = sel_ref[i * TOPN + 0]
    sel1 = sel_ref[i * TOPN + 1]
    sel2 = sel_ref[i * TOPN + 2]
    sel3 = sel_ref[i * TOPN + 3]
    blk = jnp.where(n_of_c == 0, sel0,
                    jnp.where(n_of_c == 1, sel1,
                              jnp.where(n_of_c == 2, sel2, sel3)))
    kv_pos = blk * SB + c % SB
    mask = kv_pos <= i * SB + r
    kcat = jnp.concatenate(
        [k0[...], k1[...], k2[...], k3[...]], axis=0).astype(jnp.bfloat16)
    vcat = jnp.concatenate(
        [v0[...], v1[...], v2[...], v3[...]], axis=0).astype(jnp.bfloat16)
    for h in range(H):
        qh = q_ref[:, h * D:(h + 1) * D].astype(jnp.bfloat16)
        s = jax.lax.dot_general(qh, kcat[:, h * D:(h + 1) * D], _T_DN,
                                preferred_element_type=jnp.float32) * SCALE
        s = jnp.where(mask, s, _NEG)
        m = jnp.max(s, axis=1, keepdims=True)
        e = jnp.exp(s - m)
        p = (e / jnp.sum(e, axis=1, keepdims=True)).astype(jnp.bfloat16)
        o_ref[:, h * D:(h + 1) * D] = jnp.dot(
            p, vcat[:, h * D:(h + 1) * D], preferred_element_type=jnp.float32)


def _fine(qkv, sel):
    def kmap(n):
        return lambda i, sref: (sref[i * TOPN + n], 1)

    def vmap_(n):
        return lambda i, sref: (sref[i * TOPN + n], 2)

    grid_spec = pltpu.PrefetchScalarGridSpec(
        num_scalar_prefetch=1,
        grid=(NQ,),
        in_specs=[
            pl.BlockSpec((SB, HID), lambda i, sref: (i, 0)),
            pl.BlockSpec((SB, HID), kmap(0)),
            pl.BlockSpec((SB, HID), kmap(1)),
            pl.BlockSpec((SB, HID), kmap(2)),
            pl.BlockSpec((SB, HID), kmap(3)),
            pl.BlockSpec((SB, HID), vmap_(0)),
            pl.BlockSpec((SB, HID), vmap_(1)),
            pl.BlockSpec((SB, HID), vmap_(2)),
            pl.BlockSpec((SB, HID), vmap_(3)),
        ],
        out_specs=pl.BlockSpec((SB, HID), lambda i, sref: (i, 0)),
    )
    return pl.pallas_call(
        _fine_body,
        grid_spec=grid_spec,
        out_shape=jax.ShapeDtypeStruct((S, HID), jnp.float32),
    )(sel, qkv, qkv, qkv, qkv, qkv, qkv, qkv, qkv, qkv)


def _wc_body(q_ref, kp_ref, kc_ref, vp_ref, vc_ref, oc_ref, os_ref, g_ref,
             o_ref):
    w = pl.program_id(0)
    q = q_ref[...].astype(jnp.bfloat16)
    sp = jax.lax.dot_general(q, kp_ref[...].astype(jnp.bfloat16), _T_DN,
                             preferred_element_type=jnp.float32) * SCALE
    sc_ = jax.lax.dot_general(q, kc_ref[...].astype(jnp.bfloat16), _T_DN,
                              preferred_element_type=jnp.float32) * SCALE
    r = jax.lax.broadcasted_iota(jnp.int32, (WIN, WIN), 0)
    c = jax.lax.broadcasted_iota(jnp.int32, (WIN, WIN), 1)
    sp = jnp.where((r < c) & (w > 0), sp, _NEG)
    sc_ = jnp.where(c <= r, sc_, _NEG)
    m = jnp.maximum(jnp.max(sp, axis=1, keepdims=True),
                    jnp.max(sc_, axis=1, keepdims=True))
    ep = jnp.exp(sp - m)
    ec = jnp.exp(sc_ - m)
    den = jnp.sum(ep, axis=1, keepdims=True) + jnp.sum(ec, axis=1, keepdims=True)
    pp = (ep / den).astype(jnp.bfloat16)
    pc = (ec / den).astype(jnp.bfloat16)
    ow = (jnp.dot(pp, vp_ref[...].astype(jnp.bfloat16),
                  preferred_element_type=jnp.float32)
          + jnp.dot(pc, vc_ref[...].astype(jnp.bfloat16),
                    preferred_element_type=jnp.float32))
    h = pl.program_id(1)
    g = g_ref[...]
    g = 1.0 / (1.0 + jnp.exp(-g))
    lane = jax.lax.broadcasted_iota(jnp.int32, (WIN, 3 * H), 1)

    def gcol(j):
        return jnp.sum(jnp.where(lane == 3 * h + j, g, 0.0), axis=1,
                       keepdims=True)

    o_ref[...] = (gcol(0) * oc_ref[...] + gcol(1) * os_ref[...]
                  + gcol(2) * ow)


def _win_combine(qkv, o_cmp, o_sel, graw):
    def prev_map(w, h, off):
        return (jnp.where(w == 0, 0, w - 1), off + h)

    return pl.pallas_call(
        _wc_body,
        grid=(NW, H),
        in_specs=[
            pl.BlockSpec((WIN, D), lambda w, h: (w, h)),
            pl.BlockSpec((WIN, D), lambda w, h: prev_map(w, h, H)),
            pl.BlockSpec((WIN, D), lambda w, h: (w, H + h)),
            pl.BlockSpec((WIN, D), lambda w, h: prev_map(w, h, 2 * H)),
            pl.BlockSpec((WIN, D), lambda w, h: (w, 2 * H + h)),
            pl.BlockSpec((WIN, D), lambda w, h: (w, h)),
            pl.BlockSpec((WIN, D), lambda w, h: (w, h)),
            pl.BlockSpec((WIN, 3 * H), lambda w, h: (w, 0)),
        ],
        out_specs=pl.BlockSpec((WIN, D), lambda w, h: (w, h)),
        out_shape=jax.ShapeDtypeStruct((S, HID), jnp.float32),
    )(qkv, qkv, qkv, qkv, qkv, o_cmp, o_sel, graw)


def kernel(x, Wq, Wk, Wv, Wg, Wo):
    x2 = x.reshape(S, HID)
    w3 = jnp.concatenate([Wq, Wk, Wv], axis=1)
    qkv = _matmul(x2, w3, 512, 512)
    graw = _matmul(x2, Wg, 512, 48)
    o_cmp, u = _compressed(qkv)
    selp = _topk(u)
    sel = selp[:, :TOPN].reshape(-1).astype(jnp.int32)
    o_sel = _fine(qkv, sel)
    o_cmb = _win_combine(qkv, o_cmp, o_sel, graw)
    out = _matmul(o_cmb, Wo, 256, 512)
    return out.reshape(1, S, HID)


# speedup vs baseline: 1.6602x; 1.1282x over previous
import jax
import jax.numpy as jnp
from jax.experimental import pallas as pl
from jax.experimental.pallas import tpu as pltpu

H = 16
D = 128
CB = 32
SB = 64
WIN = 512
TOPN = 4
S = 2048
HID = 2048
NC = S // CB
NS = S // SB
NQ = NS
NW = S // WIN
NPROJ = 3 * HID + 512
GBLK = (3 * HID) // (3 * H)
SCALE = 1.0 / (D ** 0.5)
_NEG = -1e9

_T_DN = (((1,), (1,)), ((), ()))


def _mm2_body(a_ref, b_ref, o_ref, obf_ref):
    o = jnp.dot(a_ref[...], b_ref[...], preferred_element_type=jnp.float32)
    o_ref[...] = o
    obf_ref[...] = o.astype(jnp.bfloat16)


def _matmul2(a, b, bn):
    m, k = a.shape
    _, n = b.shape
    return pl.pallas_call(
        _mm2_body,
        grid=(n // bn,),
        in_specs=[
            pl.BlockSpec((m, k), lambda j: (0, 0)),
            pl.BlockSpec((k, bn), lambda j: (0, j)),
        ],
        out_specs=[
            pl.BlockSpec((m, bn), lambda j: (0, j)),
            pl.BlockSpec((m, bn), lambda j: (0, j)),
        ],
        out_shape=[
            jax.ShapeDtypeStruct((m, n), jnp.float32),
            jax.ShapeDtypeStruct((m, n), jnp.bfloat16),
        ],
    )(a, b)


def _mm_body(a_ref, b_ref, o_ref):
    o_ref[...] = jnp.dot(a_ref[...], b_ref[...],
                         preferred_element_type=jnp.float32)


def _matmul(a, b, bn):
    m, k = a.shape
    _, n = b.shape
    return pl.pallas_call(
        _mm_body,
        grid=(n // bn,),
        in_specs=[
            pl.BlockSpec((m, k), lambda j: (0, 0)),
            pl.BlockSpec((k, bn), lambda j: (0, j)),
        ],
        out_specs=pl.BlockSpec((m, bn), lambda j: (0, j)),
        out_shape=jax.ShapeDtypeStruct((m, n), jnp.float32),
    )(a, b)


def _cmp_body(q_ref, k_ref, v_ref, oc_ref, u_ref, sel_ref):
    h = pl.program_id(0)
    q = q_ref[...]
    k = k_ref[...]
    v = v_ref[...]
    kc = jnp.mean(k.reshape(NC, CB, D), axis=1)
    vc = jnp.mean(v.reshape(NC, CB, D), axis=1)
    s = jax.lax.dot_general(
        q.astype(jnp.bfloat16), kc.astype(jnp.bfloat16), _T_DN,
        preferred_element_type=jnp.float32) * SCALE
    t = jax.lax.broadcasted_iota(jnp.int32, (S, NC), 0)
    n = jax.lax.broadcasted_iota(jnp.int32, (S, NC), 1)
    s = jnp.where(n * CB <= t, s, _NEG)
    m = jnp.max(s, axis=1, keepdims=True)
    e = jnp.exp(s - m)
    p = e / jnp.sum(e, axis=1, keepdims=True)
    oc_ref[...] = jnp.dot(p.astype(jnp.bfloat16), vc.astype(jnp.bfloat16),
                          preferred_element_type=jnp.float32).astype(jnp.bfloat16)
    u = jnp.sum(p.reshape(NQ, SB, NC), axis=1)

    @pl.when(h == 0)
    def _():
        u_ref[...] = u

    @pl.when(h > 0)
    def _():
        u_ref[...] += u

    @pl.when(h == H - 1)
    def _():
        ut = u_ref[...].T
        ub = jnp.sum(ut.reshape(NS, 2, NQ), axis=1)
        imp = ub.T
        r = jax.lax.broadcasted_iota(jnp.int32, (NQ, NS), 0)
        c = jax.lax.broadcasted_iota(jnp.int32, (NQ, NS), 1)
        ssc = jnp.where(c <= r, imp, -1e30)
        ssc = (ssc + 1e20 * (c == r).astype(jnp.float32)
               + 1e19 * (c == 0).astype(jnp.float32))
        out = jnp.zeros((NQ, 128), jnp.int32)
        colp = jax.lax.broadcasted_iota(jnp.int32, (NQ, 128), 1)
        for ti in range(TOPN):
            mx = jnp.max(ssc, axis=1, keepdims=True)
            idx = jnp.min(jnp.where(ssc >= mx, c, NS), axis=1, keepdims=True)
            out = out + jnp.where(colp == ti, idx, 0)
            ssc = jnp.where(c == idx, -jnp.inf, ssc)
        sel_ref[...] = out


def _compressed(qkv):
    return pl.pallas_call(
        _cmp_body,
        grid=(H,),
        in_specs=[
            pl.BlockSpec((S, D), lambda h: (0, h)),
            pl.BlockSpec((S, D), lambda h: (0, H + h)),
            pl.BlockSpec((S, D), lambda h: (0, 2 * H + h)),
        ],
        out_specs=[
            pl.BlockSpec((S, D), lambda h: (0, h)),
            pl.BlockSpec((NQ, NC), lambda h: (0, 0)),
            pl.BlockSpec((NQ, 128), lambda h: (0, 0)),
        ],
        out_shape=[
            jax.ShapeDtypeStruct((S, HID), jnp.bfloat16),
            jax.ShapeDtypeStruct((NQ, NC), jnp.float32),
            jax.ShapeDtypeStruct((NQ, 128), jnp.int32),
        ],
    )(qkv, qkv, qkv)


def _fine_body(sel_ref, q_ref, k0, k1, k2, k3, v0, v1, v2, v3, o_ref):
    i = pl.program_id(0)
    r = jax.lax.broadcasted_iota(jnp.int32, (SB, TOPN * SB), 0)
    c = jax.lax.broadcasted_iota(jnp.int32, (SB, TOPN * SB), 1)
    n_of_c = c // SB
    sel0 = sel_ref[i * TOPN + 0]
    sel1 = sel_ref[i * TOPN + 1]
    sel2 = sel_ref[i * TOPN + 2]
    sel3 = sel_ref[i * TOPN + 3]
    blk = jnp.where(n_of_c == 0, sel0,
                    jnp.where(n_of_c == 1, sel1,
                              jnp.where(n_of_c == 2, sel2, sel3)))
    kv_pos = blk * SB + c % SB
    mask = kv_pos <= i * SB + r
    kcat = jnp.concatenate([k0[...], k1[...], k2[...], k3[...]], axis=0)
    vcat = jnp.concatenate([v0[...], v1[...], v2[...], v3[...]], axis=0)
    for h in range(H):
        qh = q_ref[:, h * D:(h + 1) * D]
        s = jax.lax.dot_general(qh, kcat[:, h * D:(h + 1) * D], _T_DN,
                                preferred_element_type=jnp.float32) * SCALE
        s = jnp.where(mask, s, _NEG)
        m = jnp.max(s, axis=1, keepdims=True)
        e = jnp.exp(s - m)
        p = (e / jnp.sum(e, axis=1, keepdims=True)).astype(jnp.bfloat16)
        o_ref[:, h * D:(h + 1) * D] = jnp.dot(
            p, vcat[:, h * D:(h + 1) * D],
            preferred_element_type=jnp.float32).astype(jnp.bfloat16)


def _fine(qkv_bf, sel):
    def kmap(n):
        return lambda i, sref: (sref[i * TOPN + n], 1)

    def vmap_(n):
        return lambda i, sref: (sref[i * TOPN + n], 2)

    grid_spec = pltpu.PrefetchScalarGridSpec(
        num_scalar_prefetch=1,
        grid=(NQ,),
        in_specs=[
            pl.BlockSpec((SB, HID), lambda i, sref: (i, 0)),
            pl.BlockSpec((SB, HID), kmap(0)),
            pl.BlockSpec((SB, HID), kmap(1)),
            pl.BlockSpec((SB, HID), kmap(2)),
            pl.BlockSpec((SB, HID), kmap(3)),
            pl.BlockSpec((SB, HID), vmap_(0)),
            pl.BlockSpec((SB, HID), vmap_(1)),
            pl.BlockSpec((SB, HID), vmap_(2)),
            pl.BlockSpec((SB, HID), vmap_(3)),
        ],
        out_specs=pl.BlockSpec((SB, HID), lambda i, sref: (i, 0)),
    )
    return pl.pallas_call(
        _fine_body,
        grid_spec=grid_spec,
        out_shape=jax.ShapeDtypeStruct((S, HID), jnp.bfloat16),
    )(sel, qkv_bf, qkv_bf, qkv_bf, qkv_bf, qkv_bf, qkv_bf, qkv_bf, qkv_bf,
      qkv_bf)


def _wc_body(q_ref, kp_ref, kc_ref, vp_ref, vc_ref, oc_ref, os_ref, g_ref,
             o_ref):
    w = pl.program_id(0)
    q = q_ref[...]
    sp = jax.lax.dot_general(q, kp_ref[...], _T_DN,
                             preferred_element_type=jnp.float32) * SCALE
    sc_ = jax.lax.dot_general(q, kc_ref[...], _T_DN,
                              preferred_element_type=jnp.float32) * SCALE
    r = jax.lax.broadcasted_iota(jnp.int32, (WIN, WIN), 0)
    c = jax.lax.broadcasted_iota(jnp.int32, (WIN, WIN), 1)
    sp = jnp.where((r < c) & (w > 0), sp, _NEG)
    sc_ = jnp.where(c <= r, sc_, _NEG)
    m = jnp.maximum(jnp.max(sp, axis=1, keepdims=True),
                    jnp.max(sc_, axis=1, keepdims=True))
    ep = jnp.exp(sp - m)
    ec = jnp.exp(sc_ - m)
    den = jnp.sum(ep, axis=1, keepdims=True) + jnp.sum(ec, axis=1, keepdims=True)
    pp = (ep / den).astype(jnp.bfloat16)
    pc = (ec / den).astype(jnp.bfloat16)
    ow = (jnp.dot(pp, vp_ref[...], preferred_element_type=jnp.float32)
          + jnp.dot(pc, vc_ref[...], preferred_element_type=jnp.float32))
    h = pl.program_id(1)
    g = g_ref[...]
    g = 1.0 / (1.0 + jnp.exp(-g))
    lane = jax.lax.broadcasted_iota(jnp.int32, (WIN, 128), 1)

    def gcol(j):
        return jnp.sum(jnp.where(lane == 3 * h + j, g, 0.0), axis=1,
                       keepdims=True)

    o_ref[...] = (gcol(0) * oc_ref[...].astype(jnp.float32)
                  + gcol(1) * os_ref[...].astype(jnp.float32)
                  + gcol(2) * ow).astype(jnp.bfloat16)


def _win_combine(qkv_bf, qkv, o_cmp, o_sel):
    def prev_map(w, h, off):
        return (jnp.where(w == 0, 0, w - 1), off + h)

    return pl.pallas_call(
        _wc_body,
        grid=(NW, H),
        in_specs=[
            pl.BlockSpec((WIN, D), lambda w, h: (w, h)),
            pl.BlockSpec((WIN, D), lambda w, h: prev_map(w, h, H)),
            pl.BlockSpec((WIN, D), lambda w, h: (w, H + h)),
            pl.BlockSpec((WIN, D), lambda w, h: prev_map(w, h, 2 * H)),
            pl.BlockSpec((WIN, D), lambda w, h: (w, 2 * H + h)),
            pl.BlockSpec((WIN, D), lambda w, h: (w, h)),
            pl.BlockSpec((WIN, D), lambda w, h: (w, h)),
            pl.BlockSpec((WIN, 128), lambda w, h: (w, 3 * H)),
        ],
        out_specs=pl.BlockSpec((WIN, D), lambda w, h: (w, h)),
        out_shape=jax.ShapeDtypeStruct((S, HID), jnp.bfloat16),
    )(qkv_bf, qkv_bf, qkv_bf, qkv_bf, qkv_bf, o_cmp, o_sel, qkv)


def kernel(x, Wq, Wk, Wv, Wg, Wo):
    x2 = x.reshape(S, HID).astype(jnp.bfloat16)
    wg_pad = jnp.pad(Wg, ((0, 0), (0, 512 - 3 * H)))
    w3 = jnp.concatenate([Wq, Wk, Wv, wg_pad], axis=1).astype(jnp.bfloat16)
    qkv, qkv_bf = _matmul2(x2, w3, 512)
    o_cmp, _, selp = _compressed(qkv)
    sel = selp[:, :TOPN].reshape(-1).astype(jnp.int32)
    o_sel = _fine(qkv_bf, sel)
    o_cmb = _win_combine(qkv_bf, qkv, o_cmp, o_sel)
    out = _matmul(o_cmb, Wo.astype(jnp.bfloat16), 512)
    return out.reshape(1, S, HID)


# speedup vs baseline: 2.2965x; 1.3833x over previous
import jax
import jax.numpy as jnp
from jax.experimental import pallas as pl
from jax.experimental.pallas import tpu as pltpu

H = 16
D = 128
CB = 32
SB = 64
WIN = 512
TOPN = 4
S = 2048
HID = 2048
NC = S // CB
NS = S // SB
NQ = NS
NW = S // WIN
NPROJ = 3 * HID + 512
GBLK = (3 * HID) // (3 * H)
SCALE = 1.0 / (D ** 0.5)
_NEG = -1e9

_T_DN = (((1,), (1,)), ((), ()))


def _mm2_body(a_ref, b_ref, o_ref, obf_ref):
    o = jnp.dot(a_ref[...], b_ref[...], preferred_element_type=jnp.float32)
    o_ref[...] = o
    obf_ref[...] = o.astype(jnp.bfloat16)


def _matmul2(a, b, bn):
    m, k = a.shape
    _, n = b.shape
    return pl.pallas_call(
        _mm2_body,
        grid=(n // bn,),
        in_specs=[
            pl.BlockSpec((m, k), lambda j: (0, 0)),
            pl.BlockSpec((k, bn), lambda j: (0, j)),
        ],
        out_specs=[
            pl.BlockSpec((m, bn), lambda j: (0, j)),
            pl.BlockSpec((m, bn), lambda j: (0, j)),
        ],
        out_shape=[
            jax.ShapeDtypeStruct((m, n), jnp.float32),
            jax.ShapeDtypeStruct((m, n), jnp.bfloat16),
        ],
    )(a, b)


def _mm_body(a_ref, b_ref, o_ref):
    o_ref[...] = jnp.dot(a_ref[...], b_ref[...],
                         preferred_element_type=jnp.float32)


def _matmul(a, b, bn):
    m, k = a.shape
    _, n = b.shape
    return pl.pallas_call(
        _mm_body,
        grid=(n // bn,),
        in_specs=[
            pl.BlockSpec((m, k), lambda j: (0, 0)),
            pl.BlockSpec((k, bn), lambda j: (0, j)),
        ],
        out_specs=pl.BlockSpec((m, bn), lambda j: (0, j)),
        out_shape=jax.ShapeDtypeStruct((m, n), jnp.float32),
    )(a, b)


def _cmp_body(q_ref, k_ref, v_ref, oc_ref, u_ref, sel_ref):
    h = pl.program_id(0)
    q = q_ref[...]
    k = k_ref[...]
    v = v_ref[...]
    kc = jnp.mean(k.reshape(NC, CB, D), axis=1)
    vc = jnp.mean(v.reshape(NC, CB, D), axis=1)
    s = jax.lax.dot_general(
        q.astype(jnp.bfloat16), kc.astype(jnp.bfloat16), _T_DN,
        preferred_element_type=jnp.float32) * SCALE
    t = jax.lax.broadcasted_iota(jnp.int32, (S, NC), 0)
    n = jax.lax.broadcasted_iota(jnp.int32, (S, NC), 1)
    s = jnp.where(n * CB <= t, s, _NEG)
    m = jnp.max(s, axis=1, keepdims=True)
    e = jnp.exp(s - m)
    p = e / jnp.sum(e, axis=1, keepdims=True)
    oc_ref[...] = jnp.dot(p.astype(jnp.bfloat16), vc.astype(jnp.bfloat16),
                          preferred_element_type=jnp.float32).astype(jnp.bfloat16)
    u = jnp.sum(p.reshape(NQ, SB, NC), axis=1)

    @pl.when(h == 0)
    def _():
        u_ref[...] = u

    @pl.when(h > 0)
    def _():
        u_ref[...] += u

    @pl.when(h == H - 1)
    def _():
        ut = u_ref[...].T
        ub = jnp.sum(ut.reshape(NS, 2, NQ), axis=1)
        imp = ub.T
        r = jax.lax.broadcasted_iota(jnp.int32, (NQ, NS), 0)
        c = jax.lax.broadcasted_iota(jnp.int32, (NQ, NS), 1)
        ssc = jnp.where(c <= r, imp, -1e30)
        ssc = (ssc + 1e20 * (c == r).astype(jnp.float32)
               + 1e19 * (c == 0).astype(jnp.float32))
        out = jnp.zeros((NQ, 128), jnp.int32)
        colp = jax.lax.broadcasted_iota(jnp.int32, (NQ, 128), 1)
        for ti in range(TOPN):
            mx = jnp.max(ssc, axis=1, keepdims=True)
            idx = jnp.min(jnp.where(ssc >= mx, c, NS), axis=1, keepdims=True)
            out = out + jnp.where(colp == ti, idx, 0)
            ssc = jnp.where(c == idx, -jnp.inf, ssc)
        sel_ref[...] = out


def _compressed(qkv):
    return pl.pallas_call(
        _cmp_body,
        grid=(H,),
        in_specs=[
            pl.BlockSpec((S, D), lambda h: (0, h)),
            pl.BlockSpec((S, D), lambda h: (0, H + h)),
            pl.BlockSpec((S, D), lambda h: (0, 2 * H + h)),
        ],
        out_specs=[
            pl.BlockSpec((S, D), lambda h: (0, h)),
            pl.BlockSpec((NQ, NC), lambda h: (0, 0)),
            pl.BlockSpec((NQ, 128), lambda h: (0, 0)),
        ],
        out_shape=[
            jax.ShapeDtypeStruct((S, HID), jnp.bfloat16),
            jax.ShapeDtypeStruct((NQ, NC), jnp.float32),
            jax.ShapeDtypeStruct((NQ, 128), jnp.int32),
        ],
    )(qkv, qkv, qkv)


def _fine_body(sel_ref, q_ref, k0, k1, k2, k3, v0, v1, v2, v3, o_ref):
    i = pl.program_id(0)
    r = jax.lax.broadcasted_iota(jnp.int32, (SB, TOPN * SB), 0)
    c = jax.lax.broadcasted_iota(jnp.int32, (SB, TOPN * SB), 1)
    n_of_c = c // SB
    sel0 = sel_ref[i * TOPN + 0]
    sel1 = sel_ref[i * TOPN + 1]
    sel2 = sel_ref[i * TOPN + 2]
    sel3 = sel_ref[i * TOPN + 3]
    blk = jnp.where(n_of_c == 0, sel0,
                    jnp.where(n_of_c == 1, sel1,
                              jnp.where(n_of_c == 2, sel2, sel3)))
    kv_pos = blk * SB + c % SB
    mask = kv_pos <= i * SB + r
    kcat = jnp.concatenate([k0[...], k1[...], k2[...], k3[...]], axis=0)
    vcat = jnp.concatenate([v0[...], v1[...], v2[...], v3[...]], axis=0)
    scores = [
        jax.lax.dot_general(q_ref[:, h * D:(h + 1) * D],
                            kcat[:, h * D:(h + 1) * D], _T_DN,
                            preferred_element_type=jnp.float32) * SCALE
        for h in range(H)
    ]
    sall = jnp.concatenate(scores, axis=0)
    maskb = jnp.broadcast_to(mask[None], (H, SB, TOPN * SB)).reshape(
        H * SB, TOPN * SB)
    sall = jnp.where(maskb, sall, _NEG)
    m = jnp.max(sall, axis=1, keepdims=True)
    e = jnp.exp(sall - m)
    p = (e / jnp.sum(e, axis=1, keepdims=True)).astype(jnp.bfloat16)
    for h in range(H):
        o_ref[:, h * D:(h + 1) * D] = jnp.dot(
            p[h * SB:(h + 1) * SB, :], vcat[:, h * D:(h + 1) * D],
            preferred_element_type=jnp.float32).astype(jnp.bfloat16)


def _fine(qkv_bf, sel):
    def kmap(n):
        return lambda i, sref: (sref[i * TOPN + n], 1)

    def vmap_(n):
        return lambda i, sref: (sref[i * TOPN + n], 2)

    grid_spec = pltpu.PrefetchScalarGridSpec(
        num_scalar_prefetch=1,
        grid=(NQ,),
        in_specs=[
            pl.BlockSpec((SB, HID), lambda i, sref: (i, 0)),
            pl.BlockSpec((SB, HID), kmap(0)),
            pl.BlockSpec((SB, HID), kmap(1)),
            pl.BlockSpec((SB, HID), kmap(2)),
            pl.BlockSpec((SB, HID), kmap(3)),
            pl.BlockSpec((SB, HID), vmap_(0)),
            pl.BlockSpec((SB, HID), vmap_(1)),
            pl.BlockSpec((SB, HID), vmap_(2)),
            pl.BlockSpec((SB, HID), vmap_(3)),
        ],
        out_specs=pl.BlockSpec((SB, HID), lambda i, sref: (i, 0)),
    )
    return pl.pallas_call(
        _fine_body,
        grid_spec=grid_spec,
        out_shape=jax.ShapeDtypeStruct((S, HID), jnp.bfloat16),
    )(sel, qkv_bf, qkv_bf, qkv_bf, qkv_bf, qkv_bf, qkv_bf, qkv_bf, qkv_bf,
      qkv_bf)


def _wc_body(q_ref, kp_ref, kc_ref, vp_ref, vc_ref, oc_ref, os_ref, g_ref,
             o_ref):
    w = pl.program_id(0)
    q = q_ref[...]
    sp = jax.lax.dot_general(q, kp_ref[...], _T_DN,
                             preferred_element_type=jnp.float32) * SCALE
    sc_ = jax.lax.dot_general(q, kc_ref[...], _T_DN,
                              preferred_element_type=jnp.float32) * SCALE
    r = jax.lax.broadcasted_iota(jnp.int32, (WIN, WIN), 0)
    c = jax.lax.broadcasted_iota(jnp.int32, (WIN, WIN), 1)
    sp = jnp.where((r < c) & (w > 0), sp, _NEG)
    sc_ = jnp.where(c <= r, sc_, _NEG)
    m = jnp.maximum(jnp.max(sp, axis=1, keepdims=True),
                    jnp.max(sc_, axis=1, keepdims=True))
    ep = jnp.exp(sp - m)
    ec = jnp.exp(sc_ - m)
    den = jnp.sum(ep, axis=1, keepdims=True) + jnp.sum(ec, axis=1, keepdims=True)
    pp = (ep / den).astype(jnp.bfloat16)
    pc = (ec / den).astype(jnp.bfloat16)
    ow = (jnp.dot(pp, vp_ref[...], preferred_element_type=jnp.float32)
          + jnp.dot(pc, vc_ref[...], preferred_element_type=jnp.float32))
    h = pl.program_id(1)
    g = g_ref[...]
    g = 1.0 / (1.0 + jnp.exp(-g))
    lane = jax.lax.broadcasted_iota(jnp.int32, (WIN, 128), 1)

    def gcol(j):
        return jnp.sum(jnp.where(lane == 3 * h + j, g, 0.0), axis=1,
                       keepdims=True)

    o_ref[...] = (gcol(0) * oc_ref[...].astype(jnp.float32)
                  + gcol(1) * os_ref[...].astype(jnp.float32)
                  + gcol(2) * ow).astype(jnp.bfloat16)


def _win_combine(qkv_bf, qkv, o_cmp, o_sel):
    def prev_map(w, h, off):
        return (jnp.where(w == 0, 0, w - 1), off + h)

    return pl.pallas_call(
        _wc_body,
        grid=(NW, H),
        in_specs=[
            pl.BlockSpec((WIN, D), lambda w, h: (w, h)),
            pl.BlockSpec((WIN, D), lambda w, h: prev_map(w, h, H)),
            pl.BlockSpec((WIN, D), lambda w, h: (w, H + h)),
            pl.BlockSpec((WIN, D), lambda w, h: prev_map(w, h, 2 * H)),
            pl.BlockSpec((WIN, D), lambda w, h: (w, 2 * H + h)),
            pl.BlockSpec((WIN, D), lambda w, h: (w, h)),
            pl.BlockSpec((WIN, D), lambda w, h: (w, h)),
            pl.BlockSpec((WIN, 128), lambda w, h: (w, 3 * H)),
        ],
        out_specs=pl.BlockSpec((WIN, D), lambda w, h: (w, h)),
        out_shape=jax.ShapeDtypeStruct((S, HID), jnp.bfloat16),
    )(qkv_bf, qkv_bf, qkv_bf, qkv_bf, qkv_bf, o_cmp, o_sel, qkv)


def kernel(x, Wq, Wk, Wv, Wg, Wo):
    x2 = x.reshape(S, HID).astype(jnp.bfloat16)
    wg_pad = jnp.pad(Wg, ((0, 0), (0, 512 - 3 * H)))
    w3 = jnp.concatenate([Wq, Wk, Wv, wg_pad], axis=1).astype(jnp.bfloat16)
    qkv, qkv_bf = _matmul2(x2, w3, 512)
    o_cmp, _, selp = _compressed(qkv)
    sel = selp[:, :TOPN].reshape(-1).astype(jnp.int32)
    o_sel = _fine(qkv_bf, sel)
    o_cmb = _win_combine(qkv_bf, qkv, o_cmp, o_sel)
    out = _matmul(o_cmb, Wo.astype(jnp.bfloat16), 512)
    return out.reshape(1, S, HID)


# speedup vs baseline: 2.5966x; 1.1307x over previous
import jax
import jax.numpy as jnp
from jax.experimental import pallas as pl
from jax.experimental.pallas import tpu as pltpu

H = 16
D = 128
CB = 32
SB = 64
WIN = 512
TOPN = 4
S = 2048
HID = 2048
NC = S // CB
NS = S // SB
NQ = NS
NW = S // WIN
NPROJ = 3 * HID + 512
GBLK = (3 * HID) // (3 * H)
SCALE = 1.0 / (D ** 0.5)
_NEG = -1e9

_T_DN = (((1,), (1,)), ((), ()))


def _mm2_body(a_ref, b_ref, o_ref, obf_ref):
    o = jnp.dot(a_ref[...], b_ref[...], preferred_element_type=jnp.float32)
    o_ref[...] = o
    obf_ref[...] = o.astype(jnp.bfloat16)


def _matmul2(a, b, bn):
    m, k = a.shape
    _, n = b.shape
    return pl.pallas_call(
        _mm2_body,
        grid=(n // bn,),
        in_specs=[
            pl.BlockSpec((m, k), lambda j: (0, 0)),
            pl.BlockSpec((k, bn), lambda j: (0, j)),
        ],
        out_specs=[
            pl.BlockSpec((m, bn), lambda j: (0, j)),
            pl.BlockSpec((m, bn), lambda j: (0, j)),
        ],
        out_shape=[
            jax.ShapeDtypeStruct((m, n), jnp.float32),
            jax.ShapeDtypeStruct((m, n), jnp.bfloat16),
        ],
    )(a, b)


def _mm_body(a_ref, b_ref, o_ref):
    o_ref[...] = jnp.dot(a_ref[...], b_ref[...],
                         preferred_element_type=jnp.float32)


def _matmul(a, b, bn):
    m, k = a.shape
    _, n = b.shape
    return pl.pallas_call(
        _mm_body,
        grid=(n // bn,),
        in_specs=[
            pl.BlockSpec((m, k), lambda j: (0, 0)),
            pl.BlockSpec((k, bn), lambda j: (0, j)),
        ],
        out_specs=pl.BlockSpec((m, bn), lambda j: (0, j)),
        out_shape=jax.ShapeDtypeStruct((m, n), jnp.float32),
    )(a, b)


def _cmp_body(q_ref, k_ref, v_ref, oc_ref, u_ref, sel_ref):
    h = pl.program_id(0)
    q = q_ref[...]
    k = k_ref[...]
    v = v_ref[...]
    kc = jnp.mean(k.reshape(NC, CB, D), axis=1)
    vc = jnp.mean(v.reshape(NC, CB, D), axis=1)
    s = jax.lax.dot_general(
        q.astype(jnp.bfloat16), kc.astype(jnp.bfloat16), _T_DN,
        preferred_element_type=jnp.float32) * SCALE
    t = jax.lax.broadcasted_iota(jnp.int32, (S, NC), 0)
    n = jax.lax.broadcasted_iota(jnp.int32, (S, NC), 1)
    s = jnp.where(n * CB <= t, s, _NEG)
    m = jnp.max(s, axis=1, keepdims=True)
    e = jnp.exp(s - m)
    p = e / jnp.sum(e, axis=1, keepdims=True)
    oc_ref[...] = jnp.dot(p.astype(jnp.bfloat16), vc.astype(jnp.bfloat16),
                          preferred_element_type=jnp.float32).astype(jnp.bfloat16)
    u = jnp.sum(p.reshape(NQ, SB, NC), axis=1)

    @pl.when(h == 0)
    def _():
        u_ref[...] = u

    @pl.when(h > 0)
    def _():
        u_ref[...] += u

    @pl.when(h == H - 1)
    def _():
        ut = u_ref[...].T
        ub = jnp.sum(ut.reshape(NS, 2, NQ), axis=1)
        imp = ub.T
        r = jax.lax.broadcasted_iota(jnp.int32, (NQ, NS), 0)
        c = jax.lax.broadcasted_iota(jnp.int32, (NQ, NS), 1)
        ssc = jnp.where(c <= r, imp, -1e30)
        ssc = (ssc + 1e20 * (c == r).astype(jnp.float32)
               + 1e19 * (c == 0).astype(jnp.float32))
        out = jnp.zeros((NQ, 128), jnp.int32)
        colp = jax.lax.broadcasted_iota(jnp.int32, (NQ, 128), 1)
        for ti in range(TOPN):
            mx = jnp.max(ssc, axis=1, keepdims=True)
            idx = jnp.min(jnp.where(ssc >= mx, c, NS), axis=1, keepdims=True)
            out = out + jnp.where(colp == ti, idx, 0)
            ssc = jnp.where(c == idx, -jnp.inf, ssc)
        sel_ref[...] = out


def _compressed(qkv):
    return pl.pallas_call(
        _cmp_body,
        grid=(H,),
        in_specs=[
            pl.BlockSpec((S, D), lambda h: (0, h)),
            pl.BlockSpec((S, D), lambda h: (0, H + h)),
            pl.BlockSpec((S, D), lambda h: (0, 2 * H + h)),
        ],
        out_specs=[
            pl.BlockSpec((S, D), lambda h: (0, h)),
            pl.BlockSpec((NQ, NC), lambda h: (0, 0)),
            pl.BlockSpec((NQ, 128), lambda h: (0, 0)),
        ],
        out_shape=[
            jax.ShapeDtypeStruct((S, HID), jnp.bfloat16),
            jax.ShapeDtypeStruct((NQ, NC), jnp.float32),
            jax.ShapeDtypeStruct((NQ, 128), jnp.int32),
        ],
    )(qkv, qkv, qkv)


def _fine_body(sel_ref, q_ref, k0, k1, k2, k3, v0, v1, v2, v3, o_ref):
    i = pl.program_id(0)
    r = jax.lax.broadcasted_iota(jnp.int32, (SB, TOPN * SB), 0)
    c = jax.lax.broadcasted_iota(jnp.int32, (SB, TOPN * SB), 1)
    n_of_c = c // SB
    sel0 = sel_ref[i * TOPN + 0]
    sel1 = sel_ref[i * TOPN + 1]
    sel2 = sel_ref[i * TOPN + 2]
    sel3 = sel_ref[i * TOPN + 3]
    blk = jnp.where(n_of_c == 0, sel0,
                    jnp.where(n_of_c == 1, sel1,
                              jnp.where(n_of_c == 2, sel2, sel3)))
    kv_pos = blk * SB + c % SB
    mask = kv_pos <= i * SB + r
    kcat = jnp.concatenate([k0[...], k1[...], k2[...], k3[...]], axis=0)
    vcat = jnp.concatenate([v0[...], v1[...], v2[...], v3[...]], axis=0)
    scores = [
        jax.lax.dot_general(q_ref[:, h * D:(h + 1) * D],
                            kcat[:, h * D:(h + 1) * D], _T_DN,
                            preferred_element_type=jnp.float32) * SCALE
        for h in range(H)
    ]
    sall = jnp.concatenate(scores, axis=0)
    maskb = jnp.broadcast_to(mask[None], (H, SB, TOPN * SB)).reshape(
        H * SB, TOPN * SB)
    sall = jnp.where(maskb, sall, _NEG)
    m = jnp.max(sall, axis=1, keepdims=True)
    e = jnp.exp(sall - m)
    p = (e / jnp.sum(e, axis=1, keepdims=True)).astype(jnp.bfloat16)
    for h in range(H):
        o_ref[:, h * D:(h + 1) * D] = jnp.dot(
            p[h * SB:(h + 1) * SB, :], vcat[:, h * D:(h + 1) * D],
            preferred_element_type=jnp.float32).astype(jnp.bfloat16)


def _fine(qkv_bf, sel):
    def kmap(n):
        return lambda i, sref: (sref[i * TOPN + n], 1)

    def vmap_(n):
        return lambda i, sref: (sref[i * TOPN + n], 2)

    grid_spec = pltpu.PrefetchScalarGridSpec(
        num_scalar_prefetch=1,
        grid=(NQ,),
        in_specs=[
            pl.BlockSpec((SB, HID), lambda i, sref: (i, 0)),
            pl.BlockSpec((SB, HID), kmap(0)),
            pl.BlockSpec((SB, HID), kmap(1)),
            pl.BlockSpec((SB, HID), kmap(2)),
            pl.BlockSpec((SB, HID), kmap(3)),
            pl.BlockSpec((SB, HID), vmap_(0)),
            pl.BlockSpec((SB, HID), vmap_(1)),
            pl.BlockSpec((SB, HID), vmap_(2)),
            pl.BlockSpec((SB, HID), vmap_(3)),
        ],
        out_specs=pl.BlockSpec((SB, HID), lambda i, sref: (i, 0)),
    )
    return pl.pallas_call(
        _fine_body,
        grid_spec=grid_spec,
        out_shape=jax.ShapeDtypeStruct((S, HID), jnp.bfloat16),
    )(sel, qkv_bf, qkv_bf, qkv_bf, qkv_bf, qkv_bf, qkv_bf, qkv_bf, qkv_bf,
      qkv_bf)


H2 = 2
HG = H // H2
HW = H2 * D


def _wc_body(q_ref, kp_ref, kc_ref, vp_ref, vc_ref, oc_ref, os_ref, g_ref,
             o_ref):
    w = pl.program_id(0)
    hh = pl.program_id(1)
    r = jax.lax.broadcasted_iota(jnp.int32, (WIN, 2 * WIN), 0)
    c = jax.lax.broadcasted_iota(jnp.int32, (WIN, 2 * WIN), 1)
    mask = (((c < WIN) & (r < c) & (w > 0))
            | ((c >= WIN) & ((c - WIN) <= r)))
    maskb = jnp.broadcast_to(mask[None], (H2, WIN, 2 * WIN)).reshape(
        H2 * WIN, 2 * WIN)
    parts = []
    for he in range(H2):
        qh = q_ref[:, he * D:(he + 1) * D]
        sp = jax.lax.dot_general(qh, kp_ref[:, he * D:(he + 1) * D], _T_DN,
                                 preferred_element_type=jnp.float32)
        sc_ = jax.lax.dot_general(qh, kc_ref[:, he * D:(he + 1) * D], _T_DN,
                                  preferred_element_type=jnp.float32)
        parts.append(jnp.concatenate([sp, sc_], axis=1))
    sall = jnp.concatenate(parts, axis=0) * SCALE
    sall = jnp.where(maskb, sall, _NEG)
    m = jnp.max(sall, axis=1, keepdims=True)
    e = jnp.exp(sall - m)
    p = (e / jnp.sum(e, axis=1, keepdims=True)).astype(jnp.bfloat16)
    g = g_ref[...]
    g = 1.0 / (1.0 + jnp.exp(-g))
    lane = jax.lax.broadcasted_iota(jnp.int32, (WIN, 128), 1)
    for he in range(H2):
        ph = p[he * WIN:(he + 1) * WIN, :]
        ow = (jnp.dot(ph[:, :WIN], vp_ref[:, he * D:(he + 1) * D],
                      preferred_element_type=jnp.float32)
              + jnp.dot(ph[:, WIN:], vc_ref[:, he * D:(he + 1) * D],
                        preferred_element_type=jnp.float32))
        hglob = hh * H2 + he

        def gcol(j):
            return jnp.sum(jnp.where(lane == 3 * hglob + j, g, 0.0), axis=1,
                           keepdims=True)

        o_ref[:, he * D:(he + 1) * D] = (
            gcol(0) * oc_ref[:, he * D:(he + 1) * D].astype(jnp.float32)
            + gcol(1) * os_ref[:, he * D:(he + 1) * D].astype(jnp.float32)
            + gcol(2) * ow).astype(jnp.bfloat16)


def _win_combine(qkv_bf, qkv, o_cmp, o_sel):
    kb = (3 * HID) // HW

    def prev_map(w, hh, off):
        return (jnp.where(w == 0, 0, w - 1), off + hh)

    return pl.pallas_call(
        _wc_body,
        grid=(NW, HG),
        in_specs=[
            pl.BlockSpec((WIN, HW), lambda w, hh: (w, hh)),
            pl.BlockSpec((WIN, HW), lambda w, hh: prev_map(w, hh, HG)),
            pl.BlockSpec((WIN, HW), lambda w, hh: (w, HG + hh)),
            pl.BlockSpec((WIN, HW), lambda w, hh: prev_map(w, hh, 2 * HG)),
            pl.BlockSpec((WIN, HW), lambda w, hh: (w, 2 * HG + hh)),
            pl.BlockSpec((WIN, HW), lambda w, hh: (w, hh)),
            pl.BlockSpec((WIN, HW), lambda w, hh: (w, hh)),
            pl.BlockSpec((WIN, 128), lambda w, hh: (w, 3 * H)),
        ],
        out_specs=pl.BlockSpec((WIN, HW), lambda w, hh: (w, hh)),
        out_shape=jax.ShapeDtypeStruct((S, HID), jnp.bfloat16),
    )(qkv_bf, qkv_bf, qkv_bf, qkv_bf, qkv_bf, o_cmp, o_sel, qkv)


def kernel(x, Wq, Wk, Wv, Wg, Wo):
    x2 = x.reshape(S, HID).astype(jnp.bfloat16)
    wg_pad = jnp.pad(Wg, ((0, 0), (0, 512 - 3 * H)))
    w3 = jnp.concatenate([Wq, Wk, Wv, wg_pad], axis=1).astype(jnp.bfloat16)
    qkv, qkv_bf = _matmul2(x2, w3, 512)
    o_cmp, _, selp = _compressed(qkv)
    sel = selp[:, :TOPN].reshape(-1).astype(jnp.int32)
    o_sel = _fine(qkv_bf, sel)
    o_cmb = _win_combine(qkv_bf, qkv, o_cmp, o_sel)
    out = _matmul(o_cmb, Wo.astype(jnp.bfloat16), 512)
    return out.reshape(1, S, HID)


# speedup vs baseline: 2.6423x; 1.0176x over previous
import jax
import jax.numpy as jnp
from jax.experimental import pallas as pl
from jax.experimental.pallas import tpu as pltpu

H = 16
D = 128
CB = 32
SB = 64
WIN = 512
TOPN = 4
S = 2048
HID = 2048
NC = S // CB
NS = S // SB
NQ = NS
NW = S // WIN
NPROJ = 3 * HID + 512
GBLK = (3 * HID) // (3 * H)
SCALE = 1.0 / (D ** 0.5)
_NEG = -1e9

_T_DN = (((1,), (1,)), ((), ()))


def _mm2_body(a_ref, b_ref, o_ref, obf_ref):
    o = jnp.dot(a_ref[...], b_ref[...], preferred_element_type=jnp.float32)
    o_ref[...] = o
    obf_ref[...] = o.astype(jnp.bfloat16)


def _matmul2(a, b, bn):
    m, k = a.shape
    _, n = b.shape
    return pl.pallas_call(
        _mm2_body,
        grid=(n // bn,),
        in_specs=[
            pl.BlockSpec((m, k), lambda j: (0, 0)),
            pl.BlockSpec((k, bn), lambda j: (0, j)),
        ],
        out_specs=[
            pl.BlockSpec((m, bn), lambda j: (0, j)),
            pl.BlockSpec((m, bn), lambda j: (0, j)),
        ],
        out_shape=[
            jax.ShapeDtypeStruct((m, n), jnp.float32),
            jax.ShapeDtypeStruct((m, n), jnp.bfloat16),
        ],
    )(a, b)


def _mm_body(a_ref, b_ref, o_ref):
    o_ref[...] = jnp.dot(a_ref[...], b_ref[...],
                         preferred_element_type=jnp.float32)


def _matmul(a, b, bn):
    m, k = a.shape
    _, n = b.shape
    return pl.pallas_call(
        _mm_body,
        grid=(n // bn,),
        in_specs=[
            pl.BlockSpec((m, k), lambda j: (0, 0)),
            pl.BlockSpec((k, bn), lambda j: (0, j)),
        ],
        out_specs=pl.BlockSpec((m, bn), lambda j: (0, j)),
        out_shape=jax.ShapeDtypeStruct((m, n), jnp.float32),
    )(a, b)


def _cmp_body(q_ref, k_ref, v_ref, oc_ref, u_ref, sel_ref):
    h = pl.program_id(0)
    q = q_ref[...]
    k = k_ref[...]
    v = v_ref[...]
    kc = jnp.mean(k.reshape(NC, CB, D), axis=1)
    vc = jnp.mean(v.reshape(NC, CB, D), axis=1)
    s = jax.lax.dot_general(
        q.astype(jnp.bfloat16), kc.astype(jnp.bfloat16), _T_DN,
        preferred_element_type=jnp.float32) * SCALE
    t = jax.lax.broadcasted_iota(jnp.int32, (S, NC), 0)
    n = jax.lax.broadcasted_iota(jnp.int32, (S, NC), 1)
    s = jnp.where(n * CB <= t, s, _NEG)
    m = jnp.max(s, axis=1, keepdims=True)
    e = jnp.exp(s - m)
    p = e / jnp.sum(e, axis=1, keepdims=True)
    oc_ref[...] = jnp.dot(p.astype(jnp.bfloat16), vc.astype(jnp.bfloat16),
                          preferred_element_type=jnp.float32).astype(jnp.bfloat16)
    u = jnp.sum(p.reshape(NQ, SB, NC), axis=1)

    @pl.when(h == 0)
    def _():
        u_ref[...] = u

    @pl.when(h > 0)
    def _():
        u_ref[...] += u

    @pl.when(h == H - 1)
    def _():
        ut = u_ref[...].T
        ub = jnp.sum(ut.reshape(NS, 2, NQ), axis=1)
        imp = ub.T
        r = jax.lax.broadcasted_iota(jnp.int32, (NQ, NS), 0)
        c = jax.lax.broadcasted_iota(jnp.int32, (NQ, NS), 1)
        ssc = jnp.where(c <= r, imp, -1e30)
        ssc = (ssc + 1e20 * (c == r).astype(jnp.float32)
               + 1e19 * (c == 0).astype(jnp.float32))
        out = jnp.zeros((NQ, 128), jnp.int32)
        colp = jax.lax.broadcasted_iota(jnp.int32, (NQ, 128), 1)
        for ti in range(TOPN):
            mx = jnp.max(ssc, axis=1, keepdims=True)
            idx = jnp.min(jnp.where(ssc >= mx, c, NS), axis=1, keepdims=True)
            out = out + jnp.where(colp == ti, idx, 0)
            ssc = jnp.where(c == idx, -jnp.inf, ssc)
        sel_ref[...] = out


def _compressed(qkv):
    return pl.pallas_call(
        _cmp_body,
        grid=(H,),
        in_specs=[
            pl.BlockSpec((S, D), lambda h: (0, h)),
            pl.BlockSpec((S, D), lambda h: (0, H + h)),
            pl.BlockSpec((S, D), lambda h: (0, 2 * H + h)),
        ],
        out_specs=[
            pl.BlockSpec((S, D), lambda h: (0, h)),
            pl.BlockSpec((NQ, NC), lambda h: (0, 0)),
            pl.BlockSpec((NQ, 128), lambda h: (0, 0)),
        ],
        out_shape=[
            jax.ShapeDtypeStruct((S, HID), jnp.bfloat16),
            jax.ShapeDtypeStruct((NQ, NC), jnp.float32),
            jax.ShapeDtypeStruct((NQ, 128), jnp.int32),
        ],
    )(qkv, qkv, qkv)


def _fine_body(sel_ref, q_ref, k0, k1, k2, k3, v0, v1, v2, v3, o_ref):
    i = pl.program_id(0)
    r = jax.lax.broadcasted_iota(jnp.int32, (SB, TOPN * SB), 0)
    c = jax.lax.broadcasted_iota(jnp.int32, (SB, TOPN * SB), 1)
    n_of_c = c // SB
    sel0 = sel_ref[i * TOPN + 0]
    sel1 = sel_ref[i * TOPN + 1]
    sel2 = sel_ref[i * TOPN + 2]
    sel3 = sel_ref[i * TOPN + 3]
    blk = jnp.where(n_of_c == 0, sel0,
                    jnp.where(n_of_c == 1, sel1,
                              jnp.where(n_of_c == 2, sel2, sel3)))
    kv_pos = blk * SB + c % SB
    mask = kv_pos <= i * SB + r
    kcat = jnp.concatenate([k0[...], k1[...], k2[...], k3[...]], axis=0)
    vcat = jnp.concatenate([v0[...], v1[...], v2[...], v3[...]], axis=0)
    scores = [
        jax.lax.dot_general(q_ref[:, h * D:(h + 1) * D],
                            kcat[:, h * D:(h + 1) * D], _T_DN,
                            preferred_element_type=jnp.float32) * SCALE
        for h in range(H)
    ]
    sall = jnp.concatenate(scores, axis=0)
    maskb = jnp.broadcast_to(mask[None], (H, SB, TOPN * SB)).reshape(
        H * SB, TOPN * SB)
    sall = jnp.where(maskb, sall, _NEG)
    m = jnp.max(sall, axis=1, keepdims=True)
    e = jnp.exp(sall - m)
    p = (e / jnp.sum(e, axis=1, keepdims=True)).astype(jnp.bfloat16)
    for h in range(H):
        o_ref[:, h * D:(h + 1) * D] = jnp.dot(
            p[h * SB:(h + 1) * SB, :], vcat[:, h * D:(h + 1) * D],
            preferred_element_type=jnp.float32).astype(jnp.bfloat16)


def _fine(qkv_bf, sel):
    def kmap(n):
        return lambda i, sref: (sref[i * TOPN + n], 1)

    def vmap_(n):
        return lambda i, sref: (sref[i * TOPN + n], 2)

    grid_spec = pltpu.PrefetchScalarGridSpec(
        num_scalar_prefetch=1,
        grid=(NQ,),
        in_specs=[
            pl.BlockSpec((SB, HID), lambda i, sref: (i, 0)),
            pl.BlockSpec((SB, HID), kmap(0)),
            pl.BlockSpec((SB, HID), kmap(1)),
            pl.BlockSpec((SB, HID), kmap(2)),
            pl.BlockSpec((SB, HID), kmap(3)),
            pl.BlockSpec((SB, HID), vmap_(0)),
            pl.BlockSpec((SB, HID), vmap_(1)),
            pl.BlockSpec((SB, HID), vmap_(2)),
            pl.BlockSpec((SB, HID), vmap_(3)),
        ],
        out_specs=pl.BlockSpec((SB, HID), lambda i, sref: (i, 0)),
    )
    return pl.pallas_call(
        _fine_body,
        grid_spec=grid_spec,
        out_shape=jax.ShapeDtypeStruct((S, HID), jnp.bfloat16),
    )(sel, qkv_bf, qkv_bf, qkv_bf, qkv_bf, qkv_bf, qkv_bf, qkv_bf, qkv_bf,
      qkv_bf)


H2 = 4
HG = H // H2
HW = H2 * D


def _wc_body(q_ref, kp_ref, kc_ref, vp_ref, vc_ref, oc_ref, os_ref, g_ref,
             o_ref):
    w = pl.program_id(0)
    hh = pl.program_id(1)
    r = jax.lax.broadcasted_iota(jnp.int32, (WIN, 2 * WIN), 0)
    c = jax.lax.broadcasted_iota(jnp.int32, (WIN, 2 * WIN), 1)
    mask = (((c < WIN) & (r < c) & (w > 0))
            | ((c >= WIN) & ((c - WIN) <= r)))
    maskb = jnp.broadcast_to(mask[None], (H2, WIN, 2 * WIN)).reshape(
        H2 * WIN, 2 * WIN)
    parts = []
    for he in range(H2):
        qh = q_ref[:, he * D:(he + 1) * D]
        sp = jax.lax.dot_general(qh, kp_ref[:, he * D:(he + 1) * D], _T_DN,
                                 preferred_element_type=jnp.float32)
        sc_ = jax.lax.dot_general(qh, kc_ref[:, he * D:(he + 1) * D], _T_DN,
                                  preferred_element_type=jnp.float32)
        parts.append(jnp.concatenate([sp, sc_], axis=1))
    sall = jnp.concatenate(parts, axis=0) * SCALE
    sall = jnp.where(maskb, sall, _NEG)
    m = jnp.max(sall, axis=1, keepdims=True)
    e = jnp.exp(sall - m)
    p = (e / jnp.sum(e, axis=1, keepdims=True)).astype(jnp.bfloat16)
    g = g_ref[...]
    g = 1.0 / (1.0 + jnp.exp(-g))
    lane = jax.lax.broadcasted_iota(jnp.int32, (WIN, 128), 1)
    for he in range(H2):
        ph = p[he * WIN:(he + 1) * WIN, :]
        ow = (jnp.dot(ph[:, :WIN], vp_ref[:, he * D:(he + 1) * D],
                      preferred_element_type=jnp.float32)
              + jnp.dot(ph[:, WIN:], vc_ref[:, he * D:(he + 1) * D],
                        preferred_element_type=jnp.float32))
        hglob = hh * H2 + he

        def gcol(j):
            return jnp.sum(jnp.where(lane == 3 * hglob + j, g, 0.0), axis=1,
                           keepdims=True)

        o_ref[:, he * D:(he + 1) * D] = (
            gcol(0) * oc_ref[:, he * D:(he + 1) * D].astype(jnp.float32)
            + gcol(1) * os_ref[:, he * D:(he + 1) * D].astype(jnp.float32)
            + gcol(2) * ow).astype(jnp.bfloat16)


def _win_combine(qkv_bf, qkv, o_cmp, o_sel):
    kb = (3 * HID) // HW

    def prev_map(w, hh, off):
        return (jnp.where(w == 0, 0, w - 1), off + hh)

    return pl.pallas_call(
        _wc_body,
        grid=(NW, HG),
        in_specs=[
            pl.BlockSpec((WIN, HW), lambda w, hh: (w, hh)),
            pl.BlockSpec((WIN, HW), lambda w, hh: prev_map(w, hh, HG)),
            pl.BlockSpec((WIN, HW), lambda w, hh: (w, HG + hh)),
            pl.BlockSpec((WIN, HW), lambda w, hh: prev_map(w, hh, 2 * HG)),
            pl.BlockSpec((WIN, HW), lambda w, hh: (w, 2 * HG + hh)),
            pl.BlockSpec((WIN, HW), lambda w, hh: (w, hh)),
            pl.BlockSpec((WIN, HW), lambda w, hh: (w, hh)),
            pl.BlockSpec((WIN, 128), lambda w, hh: (w, 3 * H)),
        ],
        out_specs=pl.BlockSpec((WIN, HW), lambda w, hh: (w, hh)),
        out_shape=jax.ShapeDtypeStruct((S, HID), jnp.bfloat16),
    )(qkv_bf, qkv_bf, qkv_bf, qkv_bf, qkv_bf, o_cmp, o_sel, qkv)


def kernel(x, Wq, Wk, Wv, Wg, Wo):
    x2 = x.reshape(S, HID).astype(jnp.bfloat16)
    wg_pad = jnp.pad(Wg, ((0, 0), (0, 512 - 3 * H)))
    w3 = jnp.concatenate([Wq, Wk, Wv, wg_pad], axis=1).astype(jnp.bfloat16)
    qkv, qkv_bf = _matmul2(x2, w3, 512)
    o_cmp, _, selp = _compressed(qkv)
    sel = selp[:, :TOPN].reshape(-1).astype(jnp.int32)
    o_sel = _fine(qkv_bf, sel)
    o_cmb = _win_combine(qkv_bf, qkv, o_cmp, o_sel)
    out = _matmul(o_cmb, Wo.astype(jnp.bfloat16), 512)
    return out.reshape(1, S, HID)


# speedup vs baseline: 2.7237x; 1.0308x over previous
import jax
import jax.numpy as jnp
from jax.experimental import pallas as pl
from jax.experimental.pallas import tpu as pltpu

H = 16
D = 128
CB = 32
SB = 64
WIN = 512
TOPN = 4
S = 2048
HID = 2048
NC = S // CB
NS = S // SB
NQ = NS
NW = S // WIN
NPROJ = 3 * HID + 512
GBLK = (3 * HID) // (3 * H)
SCALE = 1.0 / (D ** 0.5)
_NEG = -1e9

_T_DN = (((1,), (1,)), ((), ()))


def _mm2_body(a_ref, b_ref, obf_ref, op_ref):
    o = jnp.dot(a_ref[...], b_ref[...], preferred_element_type=jnp.float32)
    obf_ref[...] = o.astype(jnp.bfloat16)
    op_ref[...] = jnp.mean(o.reshape(NC, CB, o.shape[1]), axis=1)


def _matmul2(a, b, bn):
    m, k = a.shape
    _, n = b.shape
    return pl.pallas_call(
        _mm2_body,
        grid=(n // bn,),
        in_specs=[
            pl.BlockSpec((m, k), lambda j: (0, 0)),
            pl.BlockSpec((k, bn), lambda j: (0, j)),
        ],
        out_specs=[
            pl.BlockSpec((m, bn), lambda j: (0, j)),
            pl.BlockSpec((NC, bn), lambda j: (0, j)),
        ],
        out_shape=[
            jax.ShapeDtypeStruct((m, n), jnp.bfloat16),
            jax.ShapeDtypeStruct((NC, n), jnp.float32),
        ],
    )(a, b)


def _mm_body(a_ref, b_ref, o_ref):
    o_ref[...] = jnp.dot(a_ref[...], b_ref[...],
                         preferred_element_type=jnp.float32)


def _matmul(a, b, bn):
    m, k = a.shape
    _, n = b.shape
    return pl.pallas_call(
        _mm_body,
        grid=(n // bn,),
        in_specs=[
            pl.BlockSpec((m, k), lambda j: (0, 0)),
            pl.BlockSpec((k, bn), lambda j: (0, j)),
        ],
        out_specs=pl.BlockSpec((m, bn), lambda j: (0, j)),
        out_shape=jax.ShapeDtypeStruct((m, n), jnp.float32),
    )(a, b)


def _cmp_body(q_ref, kc_ref, vc_ref, oc_ref, u_ref, sel_ref):
    h = pl.program_id(0)
    q = q_ref[...]
    kc = kc_ref[...]
    vc = vc_ref[...]
    s = jax.lax.dot_general(
        q, kc.astype(jnp.bfloat16), _T_DN,
        preferred_element_type=jnp.float32) * SCALE
    t = jax.lax.broadcasted_iota(jnp.int32, (S, NC), 0)
    n = jax.lax.broadcasted_iota(jnp.int32, (S, NC), 1)
    s = jnp.where(n * CB <= t, s, _NEG)
    m = jnp.max(s, axis=1, keepdims=True)
    e = jnp.exp(s - m)
    p = e / jnp.sum(e, axis=1, keepdims=True)
    oc_ref[...] = jnp.dot(p.astype(jnp.bfloat16), vc.astype(jnp.bfloat16),
                          preferred_element_type=jnp.float32).astype(jnp.bfloat16)
    u = jnp.sum(p.reshape(NQ, SB, NC), axis=1)

    @pl.when(h == 0)
    def _():
        u_ref[...] = u

    @pl.when(h > 0)
    def _():
        u_ref[...] += u

    @pl.when(h == H - 1)
    def _():
        ut = u_ref[...].T
        ub = jnp.sum(ut.reshape(NS, 2, NQ), axis=1)
        imp = ub.T
        r = jax.lax.broadcasted_iota(jnp.int32, (NQ, NS), 0)
        c = jax.lax.broadcasted_iota(jnp.int32, (NQ, NS), 1)
        ssc = jnp.where(c <= r, imp, -1e30)
        ssc = (ssc + 1e20 * (c == r).astype(jnp.float32)
               + 1e19 * (c == 0).astype(jnp.float32))
        out = jnp.zeros((NQ, 128), jnp.int32)
        colp = jax.lax.broadcasted_iota(jnp.int32, (NQ, 128), 1)
        for ti in range(TOPN):
            mx = jnp.max(ssc, axis=1, keepdims=True)
            idx = jnp.min(jnp.where(ssc >= mx, c, NS), axis=1, keepdims=True)
            out = out + jnp.where(colp == ti, idx, 0)
            ssc = jnp.where(c == idx, -jnp.inf, ssc)
        sel_ref[...] = out


def _compressed(qkv_bf, pooled):
    return pl.pallas_call(
        _cmp_body,
        grid=(H,),
        in_specs=[
            pl.BlockSpec((S, D), lambda h: (0, h)),
            pl.BlockSpec((NC, D), lambda h: (0, H + h)),
            pl.BlockSpec((NC, D), lambda h: (0, 2 * H + h)),
        ],
        out_specs=[
            pl.BlockSpec((S, D), lambda h: (0, h)),
            pl.BlockSpec((NQ, NC), lambda h: (0, 0)),
            pl.BlockSpec((NQ, 128), lambda h: (0, 0)),
        ],
        out_shape=[
            jax.ShapeDtypeStruct((S, HID), jnp.bfloat16),
            jax.ShapeDtypeStruct((NQ, NC), jnp.float32),
            jax.ShapeDtypeStruct((NQ, 128), jnp.int32),
        ],
    )(qkv_bf, pooled, pooled)


def _fine_body(sel_ref, q_ref, k0, k1, k2, k3, v0, v1, v2, v3, o_ref):
    i = pl.program_id(0)
    r = jax.lax.broadcasted_iota(jnp.int32, (SB, TOPN * SB), 0)
    c = jax.lax.broadcasted_iota(jnp.int32, (SB, TOPN * SB), 1)
    n_of_c = c // SB
    sel0 = sel_ref[i * TOPN + 0]
    sel1 = sel_ref[i * TOPN + 1]
    sel2 = sel_ref[i * TOPN + 2]
    sel3 = sel_ref[i * TOPN + 3]
    blk = jnp.where(n_of_c == 0, sel0,
                    jnp.where(n_of_c == 1, sel1,
                              jnp.where(n_of_c == 2, sel2, sel3)))
    kv_pos = blk * SB + c % SB
    mask = kv_pos <= i * SB + r
    kcat = jnp.concatenate([k0[...], k1[...], k2[...], k3[...]], axis=0)
    vcat = jnp.concatenate([v0[...], v1[...], v2[...], v3[...]], axis=0)
    scores = [
        jax.lax.dot_general(q_ref[:, h * D:(h + 1) * D],
                            kcat[:, h * D:(h + 1) * D], _T_DN,
                            preferred_element_type=jnp.float32) * SCALE
        for h in range(H)
    ]
    sall = jnp.concatenate(scores, axis=0)
    maskb = jnp.broadcast_to(mask[None], (H, SB, TOPN * SB)).reshape(
        H * SB, TOPN * SB)
    sall = jnp.where(maskb, sall, _NEG)
    m = jnp.max(sall, axis=1, keepdims=True)
    e = jnp.exp(sall - m)
    p = (e / jnp.sum(e, axis=1, keepdims=True)).astype(jnp.bfloat16)
    for h in range(H):
        o_ref[:, h * D:(h + 1) * D] = jnp.dot(
            p[h * SB:(h + 1) * SB, :], vcat[:, h * D:(h + 1) * D],
            preferred_element_type=jnp.float32).astype(jnp.bfloat16)


def _fine(qkv_bf, sel):
    def kmap(n):
        return lambda i, sref: (sref[i * TOPN + n], 1)

    def vmap_(n):
        return lambda i, sref: (sref[i * TOPN + n], 2)

    grid_spec = pltpu.PrefetchScalarGridSpec(
        num_scalar_prefetch=1,
        grid=(NQ,),
        in_specs=[
            pl.BlockSpec((SB, HID), lambda i, sref: (i, 0)),
            pl.BlockSpec((SB, HID), kmap(0)),
            pl.BlockSpec((SB, HID), kmap(1)),
            pl.BlockSpec((SB, HID), kmap(2)),
            pl.BlockSpec((SB, HID), kmap(3)),
            pl.BlockSpec((SB, HID), vmap_(0)),
            pl.BlockSpec((SB, HID), vmap_(1)),
            pl.BlockSpec((SB, HID), vmap_(2)),
            pl.BlockSpec((SB, HID), vmap_(3)),
        ],
        out_specs=pl.BlockSpec((SB, HID), lambda i, sref: (i, 0)),
    )
    return pl.pallas_call(
        _fine_body,
        grid_spec=grid_spec,
        out_shape=jax.ShapeDtypeStruct((S, HID), jnp.bfloat16),
    )(sel, qkv_bf, qkv_bf, qkv_bf, qkv_bf, qkv_bf, qkv_bf, qkv_bf, qkv_bf,
      qkv_bf)


H2 = 4
HG = H // H2
HW = H2 * D


def _wc_body(q_ref, kp_ref, kc_ref, vp_ref, vc_ref, oc_ref, os_ref, g_ref,
             o_ref):
    w = pl.program_id(0)
    hh = pl.program_id(1)
    r = jax.lax.broadcasted_iota(jnp.int32, (WIN, 2 * WIN), 0)
    c = jax.lax.broadcasted_iota(jnp.int32, (WIN, 2 * WIN), 1)
    mask = (((c < WIN) & (r < c) & (w > 0))
            | ((c >= WIN) & ((c - WIN) <= r)))
    maskb = jnp.broadcast_to(mask[None], (H2, WIN, 2 * WIN)).reshape(
        H2 * WIN, 2 * WIN)
    parts = []
    for he in range(H2):
        qh = q_ref[:, he * D:(he + 1) * D]
        sp = jax.lax.dot_general(qh, kp_ref[:, he * D:(he + 1) * D], _T_DN,
                                 preferred_element_type=jnp.float32)
        sc_ = jax.lax.dot_general(qh, kc_ref[:, he * D:(he + 1) * D], _T_DN,
                                  preferred_element_type=jnp.float32)
        parts.append(jnp.concatenate([sp, sc_], axis=1))
    sall = jnp.concatenate(parts, axis=0) * SCALE
    sall = jnp.where(maskb, sall, _NEG)
    m = jnp.max(sall, axis=1, keepdims=True)
    e = jnp.exp(sall - m)
    p = (e / jnp.sum(e, axis=1, keepdims=True)).astype(jnp.bfloat16)
    g = g_ref[...].astype(jnp.float32)
    g = 1.0 / (1.0 + jnp.exp(-g))
    lane = jax.lax.broadcasted_iota(jnp.int32, (WIN, 128), 1)
    for he in range(H2):
        ph = p[he * WIN:(he + 1) * WIN, :]
        ow = (jnp.dot(ph[:, :WIN], vp_ref[:, he * D:(he + 1) * D],
                      preferred_element_type=jnp.float32)
              + jnp.dot(ph[:, WIN:], vc_ref[:, he * D:(he + 1) * D],
                        preferred_element_type=jnp.float32))
        hglob = hh * H2 + he

        def gcol(j):
            return jnp.sum(jnp.where(lane == 3 * hglob + j, g, 0.0), axis=1,
                           keepdims=True)

        o_ref[:, he * D:(he + 1) * D] = (
            gcol(0) * oc_ref[:, he * D:(he + 1) * D].astype(jnp.float32)
            + gcol(1) * os_ref[:, he * D:(he + 1) * D].astype(jnp.float32)
            + gcol(2) * ow).astype(jnp.bfloat16)


def _win_combine(qkv_bf, o_cmp, o_sel):
    kb = (3 * HID) // HW

    def prev_map(w, hh, off):
        return (jnp.where(w == 0, 0, w - 1), off + hh)

    return pl.pallas_call(
        _wc_body,
        grid=(NW, HG),
        in_specs=[
            pl.BlockSpec((WIN, HW), lambda w, hh: (w, hh)),
            pl.BlockSpec((WIN, HW), lambda w, hh: prev_map(w, hh, HG)),
            pl.BlockSpec((WIN, HW), lambda w, hh: (w, HG + hh)),
            pl.BlockSpec((WIN, HW), lambda w, hh: prev_map(w, hh, 2 * HG)),
            pl.BlockSpec((WIN, HW), lambda w, hh: (w, 2 * HG + hh)),
            pl.BlockSpec((WIN, HW), lambda w, hh: (w, hh)),
            pl.BlockSpec((WIN, HW), lambda w, hh: (w, hh)),
            pl.BlockSpec((WIN, 128), lambda w, hh: (w, 3 * H)),
        ],
        out_specs=pl.BlockSpec((WIN, HW), lambda w, hh: (w, hh)),
        out_shape=jax.ShapeDtypeStruct((S, HID), jnp.bfloat16),
    )(qkv_bf, qkv_bf, qkv_bf, qkv_bf, qkv_bf, o_cmp, o_sel, qkv_bf)


def kernel(x, Wq, Wk, Wv, Wg, Wo):
    x2 = x.reshape(S, HID).astype(jnp.bfloat16)
    wg_pad = jnp.pad(Wg, ((0, 0), (0, 512 - 3 * H)))
    w3 = jnp.concatenate([Wq, Wk, Wv, wg_pad], axis=1).astype(jnp.bfloat16)
    qkv_bf, pooled = _matmul2(x2, w3, 512)
    o_cmp, _, selp = _compressed(qkv_bf, pooled)
    sel = selp[:, :TOPN].reshape(-1).astype(jnp.int32)
    o_sel = _fine(qkv_bf, sel)
    o_cmb = _win_combine(qkv_bf, o_cmp, o_sel)
    out = _matmul(o_cmb, Wo.astype(jnp.bfloat16), 512)
    return out.reshape(1, S, HID)
